# baseline (speedup 1.0000x reference)
import functools

import jax
import jax.numpy as jnp
from jax import lax
from jax.experimental import pallas as pl
from jax.experimental.pallas import tpu as pltpu
from jax.experimental.pallas import tpu_sc as plsc

NC = 2
NS = 16
CHUNK = 512

_CP = pltpu.CompilerParams(use_tc_tiling_on_sc=False)


def _mesh():
    return plsc.VectorSubcoreMesh(
        core_axis_name="c", subcore_axis_name="s", num_cores=NC, num_subcores=NS
    )


def _sc_degree(dst_2d, ones_c, zeros_n, np_, ep):
    kpt = ep // (NC * NS * CHUNK)
    rpt = np_ // NS

    def body(dst_hbm, ones_hbm, zeros_hbm, out_hbm, dstv, onesv, acc):
        c = lax.axis_index("c")
        s = lax.axis_index("s")
        w = c * NS + s
        pltpu.sync_copy(dst_hbm.at[pl.ds(w * kpt, kpt)], dstv)
        pltpu.sync_copy(ones_hbm, onesv)
        pltpu.sync_copy(zeros_hbm.at[pl.ds(s * rpt, rpt)],
                        acc.at[pl.ds(s * rpt, rpt)])
        plsc.subcore_barrier()

        def step(j, _):
            pltpu.sync_copy(onesv, acc.at[dstv.at[j]], add=True)
            return _

        lax.fori_loop(0, kpt, step, None)
        plsc.subcore_barrier()
        pltpu.sync_copy(acc.at[pl.ds(s * rpt, rpt)],
                        out_hbm.at[c, pl.ds(s * rpt, rpt)])

    f = pl.kernel(
        body,
        out_type=jax.ShapeDtypeStruct((NC, np_), jnp.float32),
        mesh=_mesh(), compiler_params=_CP,
        scratch_types=[
            pltpu.VMEM((kpt, CHUNK), jnp.int32),
            pltpu.VMEM((CHUNK,), jnp.float32),
            pltpu.VMEM_SHARED((np_,), jnp.float32),
        ],
    )
    return f(dst_2d, ones_c, zeros_n)


def _sc_aggregate(y, src_2d, dst_2d, zeros_2d, np_, h, ep):
    kpt = ep // (NC * NS * CHUNK)
    rpt = np_ // NS

    def body(y_hbm, src_hbm, dst_hbm, zeros_hbm, out_hbm, srcv, dstv,
             rows0, rows1, acc, sem0, sem1):
        c = lax.axis_index("c")
        s = lax.axis_index("s")
        w = c * NS + s
        pltpu.sync_copy(src_hbm.at[pl.ds(w * kpt, kpt)], srcv)
        pltpu.sync_copy(dst_hbm.at[pl.ds(w * kpt, kpt)], dstv)
        pltpu.sync_copy(zeros_hbm.at[pl.ds(s * rpt, rpt)],
                        acc.at[pl.ds(s * rpt, rpt)])
        plsc.subcore_barrier()

        pltpu.async_copy(y_hbm.at[srcv.at[0]], rows0, sem0)

        def step2(i, _):
            j0 = 2 * i
            j1 = j0 + 1
            pltpu.make_async_copy(y_hbm.at[srcv.at[j0]], rows0, sem0).wait()
            pltpu.async_copy(y_hbm.at[srcv.at[jnp.minimum(j1, kpt - 1)]],
                             rows1, sem1)
            pltpu.sync_copy(rows0, acc.at[dstv.at[j0]], add=True)
            pltpu.make_async_copy(y_hbm.at[srcv.at[j1]], rows1, sem1).wait()
            pltpu.async_copy(y_hbm.at[srcv.at[jnp.minimum(j1 + 1, kpt - 1)]],
                             rows0, sem0)
            pltpu.sync_copy(rows1, acc.at[dstv.at[j1]], add=True)
            return _

        lax.fori_loop(0, kpt // 2, step2, None)
        pltpu.make_async_copy(y_hbm.at[srcv.at[kpt - 1]], rows0, sem0).wait()
        plsc.subcore_barrier()
        pltpu.sync_copy(acc.at[pl.ds(s * rpt, rpt)],
                        out_hbm.at[c, pl.ds(s * rpt, rpt)])

    f = pl.kernel(
        body,
        out_type=jax.ShapeDtypeStruct((NC, np_, h), jnp.float32),
        mesh=_mesh(), compiler_params=_CP,
        scratch_types=[
            pltpu.VMEM((kpt, CHUNK), jnp.int32),
            pltpu.VMEM((kpt, CHUNK), jnp.int32),
            pltpu.VMEM((CHUNK, h), jnp.float32),
            pltpu.VMEM((CHUNK, h), jnp.float32),
            pltpu.VMEM_SHARED((np_, h), jnp.float32),
            pltpu.SemaphoreType.DMA,
            pltpu.SemaphoreType.DMA,
        ],
    )
    return f(y, src_2d, dst_2d, zeros_2d)


def _tc_scale1(xw, degp):
    def body(xw_ref, degp_ref, y_ref, dis_ref):
        deg = degp_ref[0, :] + degp_ref[1, :] + 1.0
        dis = lax.rsqrt(deg)[:, None]
        dis_ref[...] = dis
        y_ref[...] = xw_ref[...] * dis

    np_, h = xw.shape
    return pl.pallas_call(
        body,
        out_shape=[
            jax.ShapeDtypeStruct((np_, h), jnp.float32),
            jax.ShapeDtypeStruct((np_, 1), jnp.float32),
        ],
    )(xw, degp)


def _tc_mm(a, w):
    def body(a_ref, w_ref, o_ref):
        o_ref[...] = jnp.dot(a_ref[...], w_ref[...],
                             preferred_element_type=jnp.float32)

    m = a.shape[0]
    return pl.pallas_call(
        body,
        out_shape=jax.ShapeDtypeStruct((m, w.shape[1]), jnp.float32),
    )(a, w)


def _tc_mid(s1, y1, dis, b1, w2):
    def body(s_ref, y_ref, dis_ref, b_ref, w_ref, o_ref):
        dis = dis_ref[...]
        h1 = jnp.maximum(
            dis * (s_ref[0] + s_ref[1] + y_ref[...]) + b_ref[...], 0.0)
        o_ref[...] = jnp.dot(h1, w_ref[...],
                             preferred_element_type=jnp.float32) * dis

    np_, h = y1.shape
    return pl.pallas_call(
        body,
        out_shape=jax.ShapeDtypeStruct((np_, w2.shape[1]), jnp.float32),
    )(s1, y1, dis, b1.reshape(1, -1), w2)


def _tc_head(s2, y2, dis, b2, wp1, bp1, wp2, bp2, n):
    def body(s_ref, y_ref, dis_ref, b_ref, wp1_ref, bp1_ref, wp2_ref,
             bp2_ref, o_ref):
        dis = dis_ref[...]
        out2 = jnp.maximum(
            dis * (s_ref[0] + s_ref[1] + y_ref[...]) + b_ref[...], 0.0)
        np_ = out2.shape[0]
        mask = lax.broadcasted_iota(jnp.int32, (np_, 1), 0) < n
        emb = jnp.sum(jnp.where(mask, out2, 0.0), axis=0, keepdims=True) / n
        z = jnp.maximum(
            jnp.dot(emb, wp1_ref[...], preferred_element_type=jnp.float32)
            + bp1_ref[...], 0.0)
        raw = jnp.dot(z, wp2_ref[...],
                      preferred_element_type=jnp.float32) + bp2_ref[...]
        o_ref[...] = 2.0 + 3.0 / (1.0 + jnp.exp(-raw))

    return pl.pallas_call(
        body,
        out_shape=jax.ShapeDtypeStruct((1, wp2.shape[1]), jnp.float32),
    )(s2, y2, dis, b2.reshape(1, -1), wp1, bp1.reshape(1, -1), wp2,
      bp2.reshape(1, -1))


def _ceil_to(v, m):
    return -(-v // m) * m


@jax.jit
def kernel(x, edge_index, W1, b1, W2, b2, Wp1, bp1, Wp2, bp2):
    n, d = x.shape
    h = W1.shape[1]
    e = edge_index.shape[1]

    np_ = _ceil_to(n, NS * 16)
    ep = _ceil_to(e, NC * NS * CHUNK * 8)
    npad = np_ - n
    epad = ep - e

    x_p = jnp.pad(x, ((0, npad), (0, 0)))
    pad_idx = n + (jnp.arange(epad, dtype=jnp.int32) % jnp.int32(max(npad, 1)))
    src = jnp.concatenate([edge_index[0].astype(jnp.int32), pad_idx])
    dst = jnp.concatenate([edge_index[1].astype(jnp.int32), pad_idx])
    src_2d = src.reshape(ep // CHUNK, CHUNK)
    dst_2d = dst.reshape(ep // CHUNK, CHUNK)

    ones_c = jnp.ones((CHUNK,), jnp.float32)
    zeros_n = jnp.zeros((np_,), jnp.float32)
    zeros_2d = jnp.zeros((np_, h), jnp.float32)

    degp = _sc_degree(dst_2d, ones_c, zeros_n, np_, ep)

    xw1 = _tc_mm(x_p, W1)
    y1, dis = _tc_scale1(xw1, degp)
    s1 = _sc_aggregate(y1, src_2d, dst_2d, zeros_2d, np_, h, ep)

    y2 = _tc_mid(s1, y1, dis, b1, W2)
    s2 = _sc_aggregate(y2, src_2d, dst_2d, zeros_2d, np_, h, ep)

    return _tc_head(s2, y2, dis, b2, Wp1, bp1, Wp2, bp2, n)


# speedup vs baseline: 1.0438x; 1.0438x over previous
import functools

import jax
import jax.numpy as jnp
from jax import lax
from jax.experimental import pallas as pl
from jax.experimental.pallas import tpu as pltpu
from jax.experimental.pallas import tpu_sc as plsc

NC = 2
NS = 16
CHUNK = 256

_CP = pltpu.CompilerParams(use_tc_tiling_on_sc=False)


def _mesh():
    return plsc.VectorSubcoreMesh(
        core_axis_name="c", subcore_axis_name="s", num_cores=NC, num_subcores=NS
    )


def _sc_degree(dst_2d, ones_c, zeros_n, np_, ep):
    kpt = ep // (NC * NS * CHUNK)
    rpt = np_ // NS

    def body(dst_hbm, ones_hbm, zeros_hbm, out_hbm, dstv, onesv, acc):
        c = lax.axis_index("c")
        s = lax.axis_index("s")
        w = c * NS + s
        pltpu.sync_copy(dst_hbm.at[pl.ds(w * kpt, kpt)], dstv)
        pltpu.sync_copy(ones_hbm, onesv)
        pltpu.sync_copy(zeros_hbm.at[pl.ds(s * rpt, rpt)],
                        acc.at[pl.ds(s * rpt, rpt)])
        plsc.subcore_barrier()

        def step(j, _):
            pltpu.sync_copy(onesv, acc.at[dstv.at[j]], add=True)
            return _

        lax.fori_loop(0, kpt, step, None)
        plsc.subcore_barrier()
        pltpu.sync_copy(acc.at[pl.ds(s * rpt, rpt)],
                        out_hbm.at[c, pl.ds(s * rpt, rpt)])

    f = pl.kernel(
        body,
        out_type=jax.ShapeDtypeStruct((NC, np_), jnp.float32),
        mesh=_mesh(), compiler_params=_CP,
        scratch_types=[
            pltpu.VMEM((kpt, CHUNK), jnp.int32),
            pltpu.VMEM((CHUNK,), jnp.float32),
            pltpu.VMEM_SHARED((np_,), jnp.float32),
        ],
    )
    return f(dst_2d, ones_c, zeros_n)


def _sc_aggregate(y, src_2d, dst_2d, zeros_2d, np_, h, ep):
    kpt = ep // (NC * NS * CHUNK)
    rpt = np_ // NS

    def body(y_hbm, src_hbm, dst_hbm, zeros_hbm, out_hbm, srcv, dstv,
             rows0, rows1, acc, sem0, sem1):
        c = lax.axis_index("c")
        s = lax.axis_index("s")
        w = c * NS + s
        pltpu.sync_copy(src_hbm.at[pl.ds(w * kpt, kpt)], srcv)
        pltpu.sync_copy(dst_hbm.at[pl.ds(w * kpt, kpt)], dstv)
        pltpu.sync_copy(zeros_hbm.at[pl.ds(s * rpt, rpt)],
                        acc.at[pl.ds(s * rpt, rpt)])
        plsc.subcore_barrier()

        pltpu.async_copy(y_hbm.at[srcv.at[0]], rows0, sem0)

        def step2(i, _):
            j0 = 2 * i
            j1 = j0 + 1
            pltpu.make_async_copy(y_hbm.at[srcv.at[j0]], rows0, sem0).wait()
            pltpu.async_copy(y_hbm.at[srcv.at[jnp.minimum(j1, kpt - 1)]],
                             rows1, sem1)
            pltpu.sync_copy(rows0, acc.at[dstv.at[j0]], add=True)
            pltpu.make_async_copy(y_hbm.at[srcv.at[j1]], rows1, sem1).wait()
            pltpu.async_copy(y_hbm.at[srcv.at[jnp.minimum(j1 + 1, kpt - 1)]],
                             rows0, sem0)
            pltpu.sync_copy(rows1, acc.at[dstv.at[j1]], add=True)
            return _

        lax.fori_loop(0, kpt // 2, step2, None)
        pltpu.make_async_copy(y_hbm.at[srcv.at[kpt - 1]], rows0, sem0).wait()
        plsc.subcore_barrier()
        pltpu.sync_copy(acc.at[pl.ds(s * rpt, rpt)],
                        out_hbm.at[c, pl.ds(s * rpt, rpt)])

    f = pl.kernel(
        body,
        out_type=jax.ShapeDtypeStruct((NC, np_, h), jnp.float32),
        mesh=_mesh(), compiler_params=_CP,
        scratch_types=[
            pltpu.VMEM((kpt, CHUNK), jnp.int32),
            pltpu.VMEM((kpt, CHUNK), jnp.int32),
            pltpu.VMEM((CHUNK, h), jnp.float32),
            pltpu.VMEM((CHUNK, h), jnp.float32),
            pltpu.VMEM_SHARED((np_, h), jnp.float32),
            pltpu.SemaphoreType.DMA,
            pltpu.SemaphoreType.DMA,
        ],
    )
    return f(y, src_2d, dst_2d, zeros_2d)


def _tc_scale1(xw, degp):
    def body(xw_ref, degp_ref, y_ref, dis_ref):
        deg = degp_ref[0, :] + degp_ref[1, :] + 1.0
        dis = lax.rsqrt(deg)[:, None]
        dis_ref[...] = dis
        y_ref[...] = xw_ref[...] * dis

    np_, h = xw.shape
    return pl.pallas_call(
        body,
        out_shape=[
            jax.ShapeDtypeStruct((np_, h), jnp.float32),
            jax.ShapeDtypeStruct((np_, 1), jnp.float32),
        ],
    )(xw, degp)


def _tc_mm(a, w):
    def body(a_ref, w_ref, o_ref):
        o_ref[...] = jnp.dot(a_ref[...], w_ref[...],
                             preferred_element_type=jnp.float32)

    m = a.shape[0]
    return pl.pallas_call(
        body,
        out_shape=jax.ShapeDtypeStruct((m, w.shape[1]), jnp.float32),
    )(a, w)


def _tc_mid(s1, y1, dis, b1, w2):
    def body(s_ref, y_ref, dis_ref, b_ref, w_ref, o_ref):
        dis = dis_ref[...]
        h1 = jnp.maximum(
            dis * (s_ref[0] + s_ref[1] + y_ref[...]) + b_ref[...], 0.0)
        o_ref[...] = jnp.dot(h1, w_ref[...],
                             preferred_element_type=jnp.float32) * dis

    np_, h = y1.shape
    return pl.pallas_call(
        body,
        out_shape=jax.ShapeDtypeStruct((np_, w2.shape[1]), jnp.float32),
    )(s1, y1, dis, b1.reshape(1, -1), w2)


def _tc_head(s2, y2, dis, b2, wp1, bp1, wp2, bp2, n):
    def body(s_ref, y_ref, dis_ref, b_ref, wp1_ref, bp1_ref, wp2_ref,
             bp2_ref, o_ref):
        dis = dis_ref[...]
        out2 = jnp.maximum(
            dis * (s_ref[0] + s_ref[1] + y_ref[...]) + b_ref[...], 0.0)
        np_ = out2.shape[0]
        mask = lax.broadcasted_iota(jnp.int32, (np_, 1), 0) < n
        emb = jnp.sum(jnp.where(mask, out2, 0.0), axis=0, keepdims=True) / n
        z = jnp.maximum(
            jnp.dot(emb, wp1_ref[...], preferred_element_type=jnp.float32)
            + bp1_ref[...], 0.0)
        raw = jnp.dot(z, wp2_ref[...],
                      preferred_element_type=jnp.float32) + bp2_ref[...]
        o_ref[...] = 2.0 + 3.0 / (1.0 + jnp.exp(-raw))

    return pl.pallas_call(
        body,
        out_shape=jax.ShapeDtypeStruct((1, wp2.shape[1]), jnp.float32),
    )(s2, y2, dis, b2.reshape(1, -1), wp1, bp1.reshape(1, -1), wp2,
      bp2.reshape(1, -1))


def _ceil_to(v, m):
    return -(-v // m) * m


@jax.jit
def kernel(x, edge_index, W1, b1, W2, b2, Wp1, bp1, Wp2, bp2):
    n, d = x.shape
    h = W1.shape[1]
    e = edge_index.shape[1]

    np_ = _ceil_to(n, NS * 16)
    ep = _ceil_to(e, NC * NS * CHUNK * 8)
    npad = np_ - n
    epad = ep - e

    x_p = jnp.pad(x, ((0, npad), (0, 0)))
    pad_idx = n + (jnp.arange(epad, dtype=jnp.int32) % jnp.int32(max(npad, 1)))
    src = jnp.concatenate([edge_index[0].astype(jnp.int32), pad_idx])
    dst = jnp.concatenate([edge_index[1].astype(jnp.int32), pad_idx])
    src_2d = src.reshape(ep // CHUNK, CHUNK)
    dst_2d = dst.reshape(ep // CHUNK, CHUNK)

    ones_c = jnp.ones((CHUNK,), jnp.float32)
    zeros_n = jnp.zeros((np_,), jnp.float32)
    zeros_2d = jnp.zeros((np_, h), jnp.float32)

    degp = _sc_degree(dst_2d, ones_c, zeros_n, np_, ep)

    xw1 = _tc_mm(x_p, W1)
    y1, dis = _tc_scale1(xw1, degp)
    s1 = _sc_aggregate(y1, src_2d, dst_2d, zeros_2d, np_, h, ep)

    y2 = _tc_mid(s1, y1, dis, b1, W2)
    s2 = _sc_aggregate(y2, src_2d, dst_2d, zeros_2d, np_, h, ep)

    return _tc_head(s2, y2, dis, b2, Wp1, bp1, Wp2, bp2, n)


# speedup vs baseline: 1.0657x; 1.0210x over previous
import functools

import jax
import jax.numpy as jnp
from jax import lax
from jax.experimental import pallas as pl
from jax.experimental.pallas import tpu as pltpu
from jax.experimental.pallas import tpu_sc as plsc

NC = 2
NS = 16
CHUNK = 128
NBUF = 4

_CP = pltpu.CompilerParams(use_tc_tiling_on_sc=False)


def _mesh():
    return plsc.VectorSubcoreMesh(
        core_axis_name="c", subcore_axis_name="s", num_cores=NC, num_subcores=NS
    )


def _sc_degree(dst_2d, ones_c, zeros_n, np_, ep):
    kpt = ep // (NC * NS * CHUNK)
    rpt = np_ // NS

    def body(dst_hbm, ones_hbm, zeros_hbm, out_hbm, dstv, onesv, acc):
        c = lax.axis_index("c")
        s = lax.axis_index("s")
        w = c * NS + s
        pltpu.sync_copy(dst_hbm.at[pl.ds(w * kpt, kpt)], dstv)
        pltpu.sync_copy(ones_hbm, onesv)
        pltpu.sync_copy(zeros_hbm.at[pl.ds(s * rpt, rpt)],
                        acc.at[pl.ds(s * rpt, rpt)])
        plsc.subcore_barrier()

        def step(j, _):
            pltpu.sync_copy(onesv, acc.at[dstv.at[j]], add=True)
            return _

        lax.fori_loop(0, kpt, step, None)
        plsc.subcore_barrier()
        pltpu.sync_copy(acc.at[pl.ds(s * rpt, rpt)],
                        out_hbm.at[c, pl.ds(s * rpt, rpt)])

    f = pl.kernel(
        body,
        out_type=jax.ShapeDtypeStruct((NC, np_), jnp.float32),
        mesh=_mesh(), compiler_params=_CP,
        scratch_types=[
            pltpu.VMEM((kpt, CHUNK), jnp.int32),
            pltpu.VMEM((CHUNK,), jnp.float32),
            pltpu.VMEM_SHARED((np_,), jnp.float32),
        ],
    )
    return f(dst_2d, ones_c, zeros_n)


def _sc_aggregate(y, src_2d, dst_2d, zeros_2d, np_, h, ep):
    kpt = ep // (NC * NS * CHUNK)
    rpt = np_ // NS

    assert kpt % NBUF == 0 and kpt >= 2 * NBUF

    def body(y_hbm, src_hbm, dst_hbm, zeros_hbm, out_hbm, srcv, dstv,
             rows0, rows1, rows2, rows3, acc,
             g0, g1, g2, g3, s0, s1, s2, s3):
        rows = (rows0, rows1, rows2, rows3)
        gsem = (g0, g1, g2, g3)
        ssem = (s0, s1, s2, s3)
        c = lax.axis_index("c")
        s = lax.axis_index("s")
        w = c * NS + s
        pltpu.sync_copy(src_hbm.at[pl.ds(w * kpt, kpt)], srcv)
        pltpu.sync_copy(dst_hbm.at[pl.ds(w * kpt, kpt)], dstv)
        pltpu.sync_copy(zeros_hbm.at[pl.ds(s * rpt, rpt)],
                        acc.at[pl.ds(s * rpt, rpt)])
        plsc.subcore_barrier()

        def gather(j, b):
            pltpu.async_copy(y_hbm.at[srcv.at[j]], rows[b], gsem[b])

        def gwait(j, b):
            pltpu.make_async_copy(y_hbm.at[srcv.at[j]], rows[b], gsem[b]).wait()

        def scat(j, b):
            pltpu.async_copy(rows[b], acc.at[dstv.at[j]], ssem[b], add=True)

        def swait(j, b):
            pltpu.make_async_copy(rows[b], acc.at[dstv.at[j]], ssem[b]).wait()

        gather(0, 0)
        gather(1, 1)
        gwait(0, 0); scat(0, 0); gather(2, 2)
        gwait(1, 1); scat(1, 1); gather(3, 3)

        def step4(i, _):
            for b in range(NBUF):
                j = NBUF * i + 2 + b
                bb = (2 + b) % NBUF
                gwait(j, bb)
                scat(j, bb)
                swait(j - 2, (bb + 2) % NBUF)
                gather(j + 2, (bb + 2) % NBUF)
            return _

        lax.fori_loop(0, (kpt - 4) // NBUF, step4, None)
        jt = kpt - 2
        gwait(jt, jt % NBUF); scat(jt, jt % NBUF)
        gwait(jt + 1, (jt + 1) % NBUF); scat(jt + 1, (jt + 1) % NBUF)
        for j in range(kpt - 4, kpt):
            swait(j, j % NBUF)
        plsc.subcore_barrier()
        pltpu.sync_copy(acc.at[pl.ds(s * rpt, rpt)],
                        out_hbm.at[c, pl.ds(s * rpt, rpt)])

    f = pl.kernel(
        body,
        out_type=jax.ShapeDtypeStruct((NC, np_, h), jnp.float32),
        mesh=_mesh(), compiler_params=_CP,
        scratch_types=[
            pltpu.VMEM((kpt, CHUNK), jnp.int32),
            pltpu.VMEM((kpt, CHUNK), jnp.int32),
            pltpu.VMEM((CHUNK, h), jnp.float32),
            pltpu.VMEM((CHUNK, h), jnp.float32),
            pltpu.VMEM((CHUNK, h), jnp.float32),
            pltpu.VMEM((CHUNK, h), jnp.float32),
            pltpu.VMEM_SHARED((np_, h), jnp.float32),
        ] + [pltpu.SemaphoreType.DMA] * 8,
    )
    return f(y, src_2d, dst_2d, zeros_2d)


def _tc_scale1(xw, degp):
    def body(xw_ref, degp_ref, y_ref, dis_ref):
        deg = degp_ref[0, :] + degp_ref[1, :] + 1.0
        dis = lax.rsqrt(deg)[:, None]
        dis_ref[...] = dis
        y_ref[...] = xw_ref[...] * dis

    np_, h = xw.shape
    return pl.pallas_call(
        body,
        out_shape=[
            jax.ShapeDtypeStruct((np_, h), jnp.float32),
            jax.ShapeDtypeStruct((np_, 1), jnp.float32),
        ],
    )(xw, degp)


def _tc_mm(a, w):
    def body(a_ref, w_ref, o_ref):
        o_ref[...] = jnp.dot(a_ref[...], w_ref[...],
                             preferred_element_type=jnp.float32)

    m = a.shape[0]
    return pl.pallas_call(
        body,
        out_shape=jax.ShapeDtypeStruct((m, w.shape[1]), jnp.float32),
    )(a, w)


def _tc_mid(s1, y1, dis, b1, w2):
    def body(s_ref, y_ref, dis_ref, b_ref, w_ref, o_ref):
        dis = dis_ref[...]
        h1 = jnp.maximum(
            dis * (s_ref[0] + s_ref[1] + y_ref[...]) + b_ref[...], 0.0)
        o_ref[...] = jnp.dot(h1, w_ref[...],
                             preferred_element_type=jnp.float32) * dis

    np_, h = y1.shape
    return pl.pallas_call(
        body,
        out_shape=jax.ShapeDtypeStruct((np_, w2.shape[1]), jnp.float32),
    )(s1, y1, dis, b1.reshape(1, -1), w2)


def _tc_head(s2, y2, dis, b2, wp1, bp1, wp2, bp2, n):
    def body(s_ref, y_ref, dis_ref, b_ref, wp1_ref, bp1_ref, wp2_ref,
             bp2_ref, o_ref):
        dis = dis_ref[...]
        out2 = jnp.maximum(
            dis * (s_ref[0] + s_ref[1] + y_ref[...]) + b_ref[...], 0.0)
        np_ = out2.shape[0]
        mask = lax.broadcasted_iota(jnp.int32, (np_, 1), 0) < n
        emb = jnp.sum(jnp.where(mask, out2, 0.0), axis=0, keepdims=True) / n
        z = jnp.maximum(
            jnp.dot(emb, wp1_ref[...], preferred_element_type=jnp.float32)
            + bp1_ref[...], 0.0)
        raw = jnp.dot(z, wp2_ref[...],
                      preferred_element_type=jnp.float32) + bp2_ref[...]
        o_ref[...] = 2.0 + 3.0 / (1.0 + jnp.exp(-raw))

    return pl.pallas_call(
        body,
        out_shape=jax.ShapeDtypeStruct((1, wp2.shape[1]), jnp.float32),
    )(s2, y2, dis, b2.reshape(1, -1), wp1, bp1.reshape(1, -1), wp2,
      bp2.reshape(1, -1))


def _ceil_to(v, m):
    return -(-v // m) * m


@jax.jit
def kernel(x, edge_index, W1, b1, W2, b2, Wp1, bp1, Wp2, bp2):
    n, d = x.shape
    h = W1.shape[1]
    e = edge_index.shape[1]

    np_ = _ceil_to(n, NS * 16)
    ep = _ceil_to(e, NC * NS * CHUNK * 8)
    npad = np_ - n
    epad = ep - e

    x_p = jnp.pad(x, ((0, npad), (0, 0)))
    pad_idx = n + (jnp.arange(epad, dtype=jnp.int32) % jnp.int32(max(npad, 1)))
    src = jnp.concatenate([edge_index[0].astype(jnp.int32), pad_idx])
    dst = jnp.concatenate([edge_index[1].astype(jnp.int32), pad_idx])
    src_2d = src.reshape(ep // CHUNK, CHUNK)
    dst_2d = dst.reshape(ep // CHUNK, CHUNK)

    ones_c = jnp.ones((CHUNK,), jnp.float32)
    zeros_n = jnp.zeros((np_,), jnp.float32)
    zeros_2d = jnp.zeros((np_, h), jnp.float32)

    degp = _sc_degree(dst_2d, ones_c, zeros_n, np_, ep)

    xw1 = _tc_mm(x_p, W1)
    y1, dis = _tc_scale1(xw1, degp)
    s1 = _sc_aggregate(y1, src_2d, dst_2d, zeros_2d, np_, h, ep)

    y2 = _tc_mid(s1, y1, dis, b1, W2)
    s2 = _sc_aggregate(y2, src_2d, dst_2d, zeros_2d, np_, h, ep)

    return _tc_head(s2, y2, dis, b2, Wp1, bp1, Wp2, bp2, n)


# speedup vs baseline: 1.0677x; 1.0019x over previous
import functools

import jax
import jax.numpy as jnp
from jax import lax
from jax.experimental import pallas as pl
from jax.experimental.pallas import tpu as pltpu
from jax.experimental.pallas import tpu_sc as plsc

NC = 2
NS = 16
CHUNK = 128
NBUF = 4

_CP = pltpu.CompilerParams(use_tc_tiling_on_sc=False)


def _mesh():
    return plsc.VectorSubcoreMesh(
        core_axis_name="c", subcore_axis_name="s", num_cores=NC, num_subcores=NS
    )


def _sc_degree(dst_2d, ones_c, zeros_n, np_, ep):
    kpt = ep // (NC * NS * CHUNK)
    rpt = np_ // NS

    def body(dst_hbm, ones_hbm, zeros_hbm, out_hbm, dstv, onesv, acc):
        c = lax.axis_index("c")
        s = lax.axis_index("s")
        w = c * NS + s
        pltpu.sync_copy(dst_hbm.at[pl.ds(w * kpt, kpt)], dstv)
        pltpu.sync_copy(ones_hbm, onesv)
        pltpu.sync_copy(zeros_hbm.at[pl.ds(s * rpt, rpt)],
                        acc.at[pl.ds(s * rpt, rpt)])
        plsc.subcore_barrier()

        def step(j, _):
            pltpu.sync_copy(onesv, acc.at[dstv.at[j]], add=True)
            return _

        lax.fori_loop(0, kpt, step, None)
        plsc.subcore_barrier()
        pltpu.sync_copy(acc.at[pl.ds(s * rpt, rpt)],
                        out_hbm.at[c, pl.ds(s * rpt, rpt)])

    f = pl.kernel(
        body,
        out_type=jax.ShapeDtypeStruct((NC, np_), jnp.float32),
        mesh=_mesh(), compiler_params=_CP,
        scratch_types=[
            pltpu.VMEM((kpt, CHUNK), jnp.int32),
            pltpu.VMEM((CHUNK,), jnp.float32),
            pltpu.VMEM_SHARED((np_,), jnp.float32),
        ],
    )
    return f(dst_2d, ones_c, zeros_n)


def _sc_aggregate(y, src_2d, dst_2d, zeros_2d, np_, h, ep):
    kpt = ep // (NC * NS * CHUNK)
    rpt = np_ // NS

    assert kpt % NBUF == 0 and kpt >= 2 * NBUF

    assert rpt % CHUNK == 0

    def body(y_hbm, src_hbm, dst_hbm, zeros_hbm, out_hbm, srcv, dstv,
             rows0, rows1, rows2, rows3, acc,
             g0, g1, g2, g3, s0, s1, s2, s3):
        rows = (rows0, rows1, rows2, rows3)
        gsem = (g0, g1, g2, g3)
        ssem = (s0, s1, s2, s3)
        c = lax.axis_index("c")
        s = lax.axis_index("s")
        w = c * NS + s
        pltpu.sync_copy(src_hbm.at[pl.ds(w * kpt, kpt)], srcv)
        pltpu.sync_copy(dst_hbm.at[pl.ds(w * kpt, kpt)], dstv)
        pltpu.sync_copy(zeros_hbm, rows0)
        for k in range(rpt // CHUNK):
            pltpu.sync_copy(rows0, acc.at[pl.ds(s * rpt + k * CHUNK, CHUNK)])
        plsc.subcore_barrier()

        def gather(j, b):
            pltpu.async_copy(y_hbm.at[srcv.at[j]], rows[b], gsem[b])

        def gwait(j, b):
            pltpu.make_async_copy(y_hbm.at[srcv.at[j]], rows[b], gsem[b]).wait()

        def scat(j, b):
            pltpu.async_copy(rows[b], acc.at[dstv.at[j]], ssem[b], add=True)

        def swait(j, b):
            pltpu.make_async_copy(rows[b], acc.at[dstv.at[j]], ssem[b]).wait()

        gather(0, 0)
        gather(1, 1)
        gwait(0, 0); scat(0, 0); gather(2, 2)
        gwait(1, 1); scat(1, 1); gather(3, 3)

        def step4(i, _):
            for b in range(NBUF):
                j = NBUF * i + 2 + b
                bb = (2 + b) % NBUF
                gwait(j, bb)
                scat(j, bb)
                swait(j - 2, (bb + 2) % NBUF)
                gather(j + 2, (bb + 2) % NBUF)
            return _

        lax.fori_loop(0, (kpt - 4) // NBUF, step4, None)
        jt = kpt - 2
        gwait(jt, jt % NBUF); scat(jt, jt % NBUF)
        gwait(jt + 1, (jt + 1) % NBUF); scat(jt + 1, (jt + 1) % NBUF)
        for j in range(kpt - 4, kpt):
            swait(j, j % NBUF)
        plsc.subcore_barrier()
        pltpu.sync_copy(acc.at[pl.ds(s * rpt, rpt)],
                        out_hbm.at[c, pl.ds(s * rpt, rpt)])

    f = pl.kernel(
        body,
        out_type=jax.ShapeDtypeStruct((NC, np_, h), jnp.float32),
        mesh=_mesh(), compiler_params=_CP,
        scratch_types=[
            pltpu.VMEM((kpt, CHUNK), jnp.int32),
            pltpu.VMEM((kpt, CHUNK), jnp.int32),
            pltpu.VMEM((CHUNK, h), jnp.float32),
            pltpu.VMEM((CHUNK, h), jnp.float32),
            pltpu.VMEM((CHUNK, h), jnp.float32),
            pltpu.VMEM((CHUNK, h), jnp.float32),
            pltpu.VMEM_SHARED((np_, h), jnp.float32),
        ] + [pltpu.SemaphoreType.DMA] * 8,
    )
    return f(y, src_2d, dst_2d, zeros_2d)


def _tc_prep(x, w1, degp):
    def body(x_ref, w_ref, degp_ref, y_ref, dis_ref):
        deg = degp_ref[0, :] + degp_ref[1, :] + 1.0
        dis = lax.rsqrt(deg)[:, None]
        dis_ref[...] = dis
        xw = jnp.dot(x_ref[...], w_ref[...],
                     preferred_element_type=jnp.float32)
        y_ref[...] = xw * dis

    np_ = x.shape[0]
    h = w1.shape[1]
    return pl.pallas_call(
        body,
        out_shape=[
            jax.ShapeDtypeStruct((np_, h), jnp.float32),
            jax.ShapeDtypeStruct((np_, 1), jnp.float32),
        ],
    )(x, w1, degp)


def _tc_mid(s1, y1, dis, b1, w2):
    def body(s_ref, y_ref, dis_ref, b_ref, w_ref, o_ref):
        dis = dis_ref[...]
        h1 = jnp.maximum(
            dis * (s_ref[0] + s_ref[1] + y_ref[...]) + b_ref[...], 0.0)
        o_ref[...] = jnp.dot(h1, w_ref[...],
                             preferred_element_type=jnp.float32) * dis

    np_, h = y1.shape
    return pl.pallas_call(
        body,
        out_shape=jax.ShapeDtypeStruct((np_, w2.shape[1]), jnp.float32),
    )(s1, y1, dis, b1.reshape(1, -1), w2)


def _tc_head(s2, y2, dis, b2, wp1, bp1, wp2, bp2, n):
    def body(s_ref, y_ref, dis_ref, b_ref, wp1_ref, bp1_ref, wp2_ref,
             bp2_ref, o_ref):
        dis = dis_ref[...]
        out2 = jnp.maximum(
            dis * (s_ref[0] + s_ref[1] + y_ref[...]) + b_ref[...], 0.0)
        np_ = out2.shape[0]
        mask = lax.broadcasted_iota(jnp.int32, (np_, 1), 0) < n
        emb = jnp.sum(jnp.where(mask, out2, 0.0), axis=0, keepdims=True) / n
        z = jnp.maximum(
            jnp.dot(emb, wp1_ref[...], preferred_element_type=jnp.float32)
            + bp1_ref[...], 0.0)
        raw = jnp.dot(z, wp2_ref[...],
                      preferred_element_type=jnp.float32) + bp2_ref[...]
        o_ref[...] = 2.0 + 3.0 / (1.0 + jnp.exp(-raw))

    return pl.pallas_call(
        body,
        out_shape=jax.ShapeDtypeStruct((1, wp2.shape[1]), jnp.float32),
    )(s2, y2, dis, b2.reshape(1, -1), wp1, bp1.reshape(1, -1), wp2,
      bp2.reshape(1, -1))


def _ceil_to(v, m):
    return -(-v // m) * m


@jax.jit
def kernel(x, edge_index, W1, b1, W2, b2, Wp1, bp1, Wp2, bp2):
    n, d = x.shape
    h = W1.shape[1]
    e = edge_index.shape[1]

    np_ = _ceil_to(n, NS * 16)
    ep = _ceil_to(e, NC * NS * CHUNK * 8)
    npad = np_ - n
    epad = ep - e

    x_p = jnp.pad(x, ((0, npad), (0, 0)))
    pad_idx = n + (jnp.arange(epad, dtype=jnp.int32) % jnp.int32(max(npad, 1)))
    src = jnp.concatenate([edge_index[0].astype(jnp.int32), pad_idx])
    dst = jnp.concatenate([edge_index[1].astype(jnp.int32), pad_idx])
    src_2d = src.reshape(ep // CHUNK, CHUNK)
    dst_2d = dst.reshape(ep // CHUNK, CHUNK)

    ones_c = jnp.ones((CHUNK,), jnp.float32)
    zeros_n = jnp.zeros((np_,), jnp.float32)
    zeros_c = jnp.zeros((CHUNK, h), jnp.float32)

    degp = _sc_degree(dst_2d, ones_c, zeros_n, np_, ep)

    y1, dis = _tc_prep(x_p, W1, degp)
    s1 = _sc_aggregate(y1, src_2d, dst_2d, zeros_c, np_, h, ep)

    y2 = _tc_mid(s1, y1, dis, b1, W2)
    s2 = _sc_aggregate(y2, src_2d, dst_2d, zeros_c, np_, h, ep)

    return _tc_head(s2, y2, dis, b2, Wp1, bp1, Wp2, bp2, n)


# speedup vs baseline: 1.1273x; 1.0558x over previous
import jax
import jax.numpy as jnp
from jax import lax
from jax.experimental import pallas as pl
from jax.experimental.pallas import tpu as pltpu
from jax.experimental.pallas import tpu_sc as plsc

NC = 2
NS = 16
CHUNK = 256
ZCH = 128
NBUF = 4

_CP = pltpu.CompilerParams(use_tc_tiling_on_sc=False)


def _mesh():
    return plsc.VectorSubcoreMesh(
        core_axis_name="c", subcore_axis_name="s", num_cores=NC, num_subcores=NS
    )


def _sc_degree(dst_2d, ones_c, zeros_n, np_, ep):
    kpt = ep // (NC * NS * CHUNK)
    rpt = np_ // NS

    def body(dst_hbm, ones_hbm, zeros_hbm, out_hbm, dstv, onesv, acc):
        c = lax.axis_index("c")
        s = lax.axis_index("s")
        w = c * NS + s
        pltpu.sync_copy(dst_hbm.at[pl.ds(w * kpt, kpt)], dstv)
        pltpu.sync_copy(ones_hbm, onesv)
        pltpu.sync_copy(zeros_hbm.at[pl.ds(s * rpt, rpt)],
                        acc.at[pl.ds(s * rpt, rpt)])
        plsc.subcore_barrier()

        def step(j, _):
            pltpu.sync_copy(onesv, acc.at[dstv.at[j]], add=True)
            return _

        lax.fori_loop(0, kpt, step, None)
        plsc.subcore_barrier()
        pltpu.sync_copy(acc.at[pl.ds(s * rpt, rpt)],
                        out_hbm.at[c, pl.ds(s * rpt, rpt)])

    f = pl.kernel(
        body,
        out_type=jax.ShapeDtypeStruct((NC, np_), jnp.float32),
        mesh=_mesh(), compiler_params=_CP,
        scratch_types=[
            pltpu.VMEM((kpt, CHUNK), jnp.int32),
            pltpu.VMEM((CHUNK,), jnp.float32),
            pltpu.VMEM_SHARED((np_,), jnp.float32),
        ],
    )
    return f(dst_2d, ones_c, zeros_n)


def _sc_aggregate(y, src_2d, dst_2d, zeros_2d, np_, h, ep):
    kpt = ep // (NC * NS * CHUNK)
    rpt = np_ // NS

    assert kpt % NBUF == 0 and kpt >= 2 * NBUF

    assert rpt % ZCH == 0

    def body(y_hbm, src_hbm, dst_hbm, zeros_hbm, out_hbm, srcv, dstv,
             rows0, rows1, rows2, rows3, acc,
             g0, g1, g2, g3, s0, s1, s2, s3):
        rows = (rows0, rows1, rows2, rows3)
        gsem = (g0, g1, g2, g3)
        ssem = (s0, s1, s2, s3)
        c = lax.axis_index("c")
        s = lax.axis_index("s")
        w = c * NS + s
        pltpu.sync_copy(src_hbm.at[pl.ds(w * kpt, kpt)], srcv)
        pltpu.sync_copy(dst_hbm.at[pl.ds(w * kpt, kpt)], dstv)
        pltpu.sync_copy(zeros_hbm, rows0.at[pl.ds(0, ZCH)])
        for k in range(rpt // ZCH):
            pltpu.sync_copy(rows0.at[pl.ds(0, ZCH)],
                            acc.at[pl.ds(s * rpt + k * ZCH, ZCH)])
        plsc.subcore_barrier()

        def gather(j, b):
            pltpu.async_copy(y_hbm.at[srcv.at[j]], rows[b], gsem[b])

        def gwait(j, b):
            pltpu.make_async_copy(y_hbm.at[srcv.at[j]], rows[b], gsem[b]).wait()

        def scat(j, b):
            pltpu.async_copy(rows[b], acc.at[dstv.at[j]], ssem[b], add=True)

        def swait(j, b):
            pltpu.make_async_copy(rows[b], acc.at[dstv.at[j]], ssem[b]).wait()

        gather(0, 0)
        gather(1, 1)
        gwait(0, 0); scat(0, 0); gather(2, 2)
        gwait(1, 1); scat(1, 1); gather(3, 3)

        def step4(i, _):
            for b in range(NBUF):
                j = NBUF * i + 2 + b
                bb = (2 + b) % NBUF
                gwait(j, bb)
                scat(j, bb)
                swait(j - 2, (bb + 2) % NBUF)
                gather(j + 2, (bb + 2) % NBUF)
            return _

        lax.fori_loop(0, (kpt - 4) // NBUF, step4, None)
        jt = kpt - 2
        gwait(jt, jt % NBUF); scat(jt, jt % NBUF)
        gwait(jt + 1, (jt + 1) % NBUF); scat(jt + 1, (jt + 1) % NBUF)
        for j in range(kpt - 4, kpt):
            swait(j, j % NBUF)
        plsc.subcore_barrier()
        pltpu.sync_copy(acc.at[pl.ds(s * rpt, rpt)],
                        out_hbm.at[c, pl.ds(s * rpt, rpt)])

    f = pl.kernel(
        body,
        out_type=jax.ShapeDtypeStruct((NC, np_, h), jnp.float32),
        mesh=_mesh(), compiler_params=_CP,
        scratch_types=[
            pltpu.VMEM((kpt, CHUNK), jnp.int32),
            pltpu.VMEM((kpt, CHUNK), jnp.int32),
            pltpu.VMEM((CHUNK, h), jnp.float32),
            pltpu.VMEM((CHUNK, h), jnp.float32),
            pltpu.VMEM((CHUNK, h), jnp.float32),
            pltpu.VMEM((CHUNK, h), jnp.float32),
            pltpu.VMEM_SHARED((np_, h), jnp.float32),
        ] + [pltpu.SemaphoreType.DMA] * 8,
    )
    return f(y, src_2d, dst_2d, zeros_2d)


def _tc_prep(x, w1, degp):
    def body(x_ref, w_ref, degp_ref, y_ref, dis_ref):
        deg = degp_ref[0, :] + degp_ref[1, :] + 1.0
        dis = lax.rsqrt(deg)[:, None]
        dis_ref[...] = dis
        xw = jnp.dot(x_ref[...], w_ref[...],
                     preferred_element_type=jnp.float32)
        y_ref[...] = xw * dis

    np_ = x.shape[0]
    h = w1.shape[1]
    return pl.pallas_call(
        body,
        out_shape=[
            jax.ShapeDtypeStruct((np_, h), jnp.float32),
            jax.ShapeDtypeStruct((np_, 1), jnp.float32),
        ],
    )(x, w1, degp)


def _tc_mid(s1, y1, dis, b1, w2):
    def body(s_ref, y_ref, dis_ref, b_ref, w_ref, o_ref):
        dis = dis_ref[...]
        h1 = jnp.maximum(
            dis * (s_ref[0] + s_ref[1] + y_ref[...]) + b_ref[...], 0.0)
        o_ref[...] = jnp.dot(h1, w_ref[...],
                             preferred_element_type=jnp.float32) * dis

    np_, h = y1.shape
    return pl.pallas_call(
        body,
        out_shape=jax.ShapeDtypeStruct((np_, w2.shape[1]), jnp.float32),
    )(s1, y1, dis, b1.reshape(1, -1), w2)


def _tc_head(s2, y2, dis, b2, wp1, bp1, wp2, bp2, n):
    def body(s_ref, y_ref, dis_ref, b_ref, wp1_ref, bp1_ref, wp2_ref,
             bp2_ref, o_ref):
        dis = dis_ref[...]
        out2 = jnp.maximum(
            dis * (s_ref[0] + s_ref[1] + y_ref[...]) + b_ref[...], 0.0)
        np_ = out2.shape[0]
        mask = lax.broadcasted_iota(jnp.int32, (np_, 1), 0) < n
        emb = jnp.sum(jnp.where(mask, out2, 0.0), axis=0, keepdims=True) / n
        z = jnp.maximum(
            jnp.dot(emb, wp1_ref[...], preferred_element_type=jnp.float32)
            + bp1_ref[...], 0.0)
        raw = jnp.dot(z, wp2_ref[...],
                      preferred_element_type=jnp.float32) + bp2_ref[...]
        o_ref[...] = 2.0 + 3.0 / (1.0 + jnp.exp(-raw))

    return pl.pallas_call(
        body,
        out_shape=jax.ShapeDtypeStruct((1, wp2.shape[1]), jnp.float32),
    )(s2, y2, dis, b2.reshape(1, -1), wp1, bp1.reshape(1, -1), wp2,
      bp2.reshape(1, -1))


def _ceil_to(v, m):
    return -(-v // m) * m


@jax.jit
def kernel(x, edge_index, W1, b1, W2, b2, Wp1, bp1, Wp2, bp2):
    n, d = x.shape
    h = W1.shape[1]
    e = edge_index.shape[1]

    np_ = _ceil_to(n, NS * 16)
    ep = _ceil_to(e, NC * NS * CHUNK * 8)
    npad = np_ - n
    epad = ep - e

    x_p = jnp.pad(x, ((0, npad), (0, 0)))
    pad_idx = n + (jnp.arange(epad, dtype=jnp.int32) % jnp.int32(max(npad, 1)))
    src = jnp.concatenate([edge_index[0].astype(jnp.int32), pad_idx])
    dst = jnp.concatenate([edge_index[1].astype(jnp.int32), pad_idx])
    src_2d = src.reshape(ep // CHUNK, CHUNK)
    dst_2d = dst.reshape(ep // CHUNK, CHUNK)

    ones_c = jnp.ones((CHUNK,), jnp.float32)
    zeros_n = jnp.zeros((np_,), jnp.float32)
    zeros_c = jnp.zeros((ZCH, h), jnp.float32)

    degp = _sc_degree(dst_2d, ones_c, zeros_n, np_, ep)

    y1, dis = _tc_prep(x_p, W1, degp)
    s1 = _sc_aggregate(y1, src_2d, dst_2d, zeros_c, np_, h, ep)

    y2 = _tc_mid(s1, y1, dis, b1, W2)
    s2 = _sc_aggregate(y2, src_2d, dst_2d, zeros_c, np_, h, ep)

    return _tc_head(s2, y2, dis, b2, Wp1, bp1, Wp2, bp2, n)
